# baseline (speedup 1.0000x reference)
import jax
import jax.numpy as jnp
from jax.experimental import pallas as pl
from jax.experimental.pallas import tpu as pltpu


def _gate_scale_kernel(idx_ref, lw_ref, feat_ref, out_ref):
    w = lw_ref[0, idx_ref[0]]
    gate = 1.0 / (1.0 + jnp.exp(-w))
    out_ref[...] = feat_ref[...] * gate


def kernel(idx, feat, layerweight):
    n0, n1, n2, n3 = feat.shape
    feat_t = jnp.transpose(feat, (0, 2, 3, 1))
    b0 = 2 if n0 % 2 == 0 else 1
    block = (b0, n2, n3, n1)
    idx_arr = jnp.asarray(idx, dtype=jnp.int32).reshape((1,))
    out_t = pl.pallas_call(
        _gate_scale_kernel,
        grid=(n0 // b0,),
        in_specs=[
            pl.BlockSpec(memory_space=pltpu.SMEM),
            pl.BlockSpec(memory_space=pltpu.SMEM),
            pl.BlockSpec(block, lambda i: (i, 0, 0, 0)),
        ],
        out_specs=pl.BlockSpec(block, lambda i: (i, 0, 0, 0)),
        out_shape=jax.ShapeDtypeStruct((n0, n2, n3, n1), feat.dtype),
    )(idx_arr, layerweight, feat_t)
    return jnp.transpose(out_t, (0, 3, 1, 2))


# speedup vs baseline: 1.0376x; 1.0376x over previous
import jax
import jax.numpy as jnp
from jax.experimental import pallas as pl
from jax.experimental.pallas import tpu as pltpu


def _gate_scale_kernel(idx_ref, lw_ref, feat_ref, out_ref):
    w = lw_ref[0, idx_ref[0]]
    gate = 1.0 / (1.0 + jnp.exp(-w))
    out_ref[...] = feat_ref[...] * gate


def kernel(idx, feat, layerweight):
    n0, n1, n2, n3 = feat.shape
    feat_t = jnp.transpose(feat, (0, 2, 3, 1))
    b0 = 8 if n0 % 8 == 0 else 1
    block = (b0, n2, n3, n1)
    idx_arr = jnp.asarray(idx, dtype=jnp.int32).reshape((1,))
    out_t = pl.pallas_call(
        _gate_scale_kernel,
        grid=(n0 // b0,),
        in_specs=[
            pl.BlockSpec(memory_space=pltpu.SMEM),
            pl.BlockSpec(memory_space=pltpu.SMEM),
            pl.BlockSpec(block, lambda i: (i, 0, 0, 0)),
        ],
        out_specs=pl.BlockSpec(block, lambda i: (i, 0, 0, 0)),
        out_shape=jax.ShapeDtypeStruct((n0, n2, n3, n1), feat.dtype),
        compiler_params=pltpu.CompilerParams(
            dimension_semantics=("arbitrary",),
            vmem_limit_bytes=128 * 1024 * 1024,
        ),
    )(idx_arr, layerweight, feat_t)
    return jnp.transpose(out_t, (0, 3, 1, 2))
